# baseline (speedup 1.0000x reference)
import functools

import jax
import jax.numpy as jnp
from jax import lax
from jax.experimental import pallas as pl
from jax.experimental.pallas import tpu as pltpu
from jax.experimental.pallas import tpu_sc as plsc

_MAX_ATOMIC_NUM = 84
_EMBED = 256
_NUM_MESSAGES = 6
_RBF_DIM = 128
_B, _N, _E = 8, 128, 1024
_NA = _B * _N
_NE = _B * _E
_EBLK = 512
_NBLK = _NE // _EBLK

_NC, _NS = 2, 16
_NW = _NC * _NS


def _sc_gather(table, idx2, D, B, dtype=jnp.float32):
    b_per_w = B // _NW
    ch = idx2.shape[1]
    n_ch = b_per_w // ch
    mesh = plsc.VectorSubcoreMesh(core_axis_name="c", subcore_axis_name="s")

    @functools.partial(
        pl.kernel, mesh=mesh,
        out_type=jax.ShapeDtypeStruct((B, D), dtype),
        scratch_types=[
            pltpu.VMEM((n_ch, ch), jnp.int32),
            pltpu.VMEM((2, ch, D), dtype),
            pltpu.SemaphoreType.DMA,
            pltpu.SemaphoreType.DMA,
        ],
    )
    def k(table_hbm, idx_hbm, out_hbm, idx_v, rows_v, gsem, ssem):
        wid = lax.axis_index("s") * _NC + lax.axis_index("c")
        base = wid * b_per_w
        pltpu.sync_copy(idx_hbm.at[pl.ds(wid * n_ch, n_ch)], idx_v)
        gathers = [None] * n_ch
        stores = [None] * n_ch
        gathers[0] = pltpu.async_copy(table_hbm.at[idx_v.at[0]],
                                      rows_v.at[0], gsem)
        for j in range(n_ch):
            b = j % 2
            if j + 1 < n_ch:
                if j - 1 >= 0:
                    stores[j - 1].wait()
                gathers[j + 1] = pltpu.async_copy(
                    table_hbm.at[idx_v.at[j + 1]], rows_v.at[1 - b], gsem)
            gathers[j].wait()
            stores[j] = pltpu.async_copy(
                rows_v.at[b], out_hbm.at[pl.ds(base + j * ch, ch)], ssem)
        for j in range(max(0, n_ch - 2), n_ch):
            stores[j].wait()

    return k(table, idx2)


def _bond_init_body(dist_ref, cent_ref, gap_ref, bondW_ref, bondb_ref,
                    out_ref):
    f32 = jnp.float32
    d = dist_ref[:]
    d = jnp.where(d != d, jnp.zeros_like(d), d)
    off = d - cent_ref[0:1, :]
    rbf = jnp.exp(-gap_ref[0:1, :] * off * off)
    b0 = jnp.dot(rbf, bondW_ref[:], preferred_element_type=f32)
    out_ref[:] = b0 + bondb_ref[0:1, :]


@jax.jit
def _tc_bond_init(dist_flat, cent, gap, bondW, bondb):
    eblk = pl.BlockSpec((_EBLK, 1), lambda b: (b, 0))
    return pl.pallas_call(
        _bond_init_body,
        grid=(_NBLK,),
        in_specs=[
            eblk,
            pl.BlockSpec((8, 128), lambda b: (0, 0)),
            pl.BlockSpec((8, 128), lambda b: (0, 0)),
            pl.BlockSpec((_RBF_DIM, _EMBED), lambda b: (0, 0)),
            pl.BlockSpec((8, _EMBED), lambda b: (0, 0)),
        ],
        out_specs=pl.BlockSpec((_EBLK, _EMBED), lambda b: (b, 0)),
        out_shape=jax.ShapeDtypeStruct((_NE, _EMBED), jnp.float32),
    )(dist_flat, cent, gap, bondW, bondb)


def _unpack(p):
    f32 = jnp.float32
    hi = jax.lax.bitcast_convert_type(
        jnp.bitwise_and(p, jnp.int32(-65536)), f32)
    lo = jax.lax.bitcast_convert_type(jnp.left_shift(p, 16), f32)
    return jnp.concatenate([hi, lo], axis=1)


def _pack(x):
    i32 = jnp.int32
    bits = jax.lax.bitcast_convert_type(x, i32)
    rnd = jnp.bitwise_and(bits + jnp.int32(0x8000), jnp.int32(-65536))
    hi = rnd[:, :128]
    lo = jax.lax.shift_right_logical(rnd[:, 128:], 16)
    return jnp.bitwise_or(hi, lo)


def _step_body(bond_ref, src_ref, dst_ref, dist_ref, dstT_ref,
               eW1b_ref, eW1sd_ref, eb1_ref, eW2_ref, eb2_ref,
               nW1s_ref, nW1b_ref, nb1_ref, nW2_ref, nb2_ref,
               bond_out_ref, agg_out_ref):
    f32 = jnp.float32
    bf16 = jnp.bfloat16
    blk = pl.program_id(0)
    bond = bond_ref[:]
    src_bf = _unpack(src_ref[:]).astype(bf16)
    dst_bf = _unpack(dst_ref[:]).astype(bf16)
    h = (jnp.dot(bond, eW1b_ref[:], preferred_element_type=f32)
         + jnp.dot(src_bf, eW1sd_ref[:_EMBED, :], preferred_element_type=f32)
         + jnp.dot(dst_bf, eW1sd_ref[_EMBED:, :], preferred_element_type=f32)
         + eb1_ref[0:1, :])
    h = jnp.maximum(h, 0.0)
    h = jnp.dot(h, eW2_ref[:], preferred_element_type=f32) + eb2_ref[0:1, :]
    bond_new = bond + h
    bond_out_ref[:] = bond_new
    m = (jnp.dot(src_bf, nW1s_ref[:], preferred_element_type=f32)
         + jnp.dot(bond_new, nW1b_ref[:], preferred_element_type=f32)
         + nb1_ref[0:1, :])
    m = jnp.maximum(m, 0.0)
    m = jnp.dot(m, nW2_ref[:], preferred_element_type=f32) + nb2_ref[0:1, :]
    dmask = (dist_ref[:] != 0.0).astype(f32)
    m = m * dmask

    row_i = jax.lax.broadcasted_iota(jnp.int32, (_NA, _EBLK), 0)
    ohT = (row_i == dstT_ref[0]).astype(jnp.bfloat16)
    contrib = jnp.dot(ohT, m.astype(jnp.bfloat16), preferred_element_type=f32)

    @pl.when(blk == 0)
    def _set():
        agg_out_ref[:] = contrib

    @pl.when(blk != 0)
    def _add():
        agg_out_ref[:] = agg_out_ref[:] + contrib


@jax.jit
def _tc_step(bond, gath, dist_flat, dstT,
             eW1b, eW1sd, eb1, eW2, eb2, nW1s, nW1b, nb1, nW2, nb2):
    eblk = pl.BlockSpec((_EBLK, _EMBED), lambda b: (b, 0))
    c2 = lambda shp: pl.BlockSpec(shp, lambda b: (0, 0))
    return pl.pallas_call(
        _step_body,
        grid=(_NBLK,),
        in_specs=[
            eblk,
            pl.BlockSpec((_EBLK, 128), lambda b: (b, 0)),
            pl.BlockSpec((_EBLK, 128), lambda b: (b + _NBLK, 0)),
            pl.BlockSpec((_EBLK, 1), lambda b: (b, 0)),
            pl.BlockSpec((1, 1, _EBLK), lambda b: (b, 0, 0)),
            c2((_EMBED, 2 * _EMBED)),
            c2((2 * _EMBED, 2 * _EMBED)),
            c2((8, 2 * _EMBED)),
            c2((2 * _EMBED, _EMBED)), c2((8, _EMBED)),
            c2((_EMBED, 2 * _EMBED)),
            c2((_EMBED, 2 * _EMBED)),
            c2((8, 2 * _EMBED)),
            c2((2 * _EMBED, _EMBED)), c2((8, _EMBED)),
        ],
        out_specs=[eblk, pl.BlockSpec((_NA, _EMBED), lambda b: (0, 0))],
        out_shape=[jax.ShapeDtypeStruct((_NE, _EMBED), jnp.float32),
                   jax.ShapeDtypeStruct((_NA, _EMBED), jnp.float32)],
        input_output_aliases={0: 0},
    )(bond, gath, gath, dist_flat, dstT,
      eW1b, eW1sd, eb1, eW2, eb2, nW1s, nW1b, nb1, nW2, nb2)


def _node_body(agg_ref, atom_ref, nW3_ref, nb3_ref, nW4_ref, nb4_ref,
               atom_out_ref, atom_pk_out_ref):
    f32 = jnp.float32
    a = jnp.dot(agg_ref[:], nW3_ref[:], preferred_element_type=f32) + nb3_ref[0:1, :]
    a = jnp.maximum(a, 0.0)
    a = jnp.dot(a, nW4_ref[:], preferred_element_type=f32) + nb4_ref[0:1, :]
    new_atom = atom_ref[:] + a
    atom_out_ref[:] = new_atom
    atom_pk_out_ref[:] = _pack(new_atom)


@jax.jit
def _tc_node(agg, atom, nW3, nb3, nW4, nb4):
    c0 = lambda shp: pl.BlockSpec(shp, lambda: tuple(0 for _ in shp))
    return pl.pallas_call(
        _node_body,
        in_specs=[
            c0((_NA, _EMBED)),
            c0((_NA, _EMBED)),
            c0((_EMBED, 2 * _EMBED)), c0((8, 2 * _EMBED)),
            c0((2 * _EMBED, _EMBED)), c0((8, _EMBED)),
        ],
        out_specs=[c0((_NA, _EMBED)), c0((_NA, 128))],
        out_shape=[jax.ShapeDtypeStruct((_NA, _EMBED), jnp.float32),
                   jax.ShapeDtypeStruct((_NA, 128), jnp.int32)],
        input_output_aliases={1: 0},
    )(agg, atom, nW3, nb3, nW4, nb4)


def _readout_body(atom_ref, meanoff_ref, site_ref, offW_ref, out_ref):
    f32 = jnp.float32
    offv = jnp.dot(atom_ref[:], offW_ref[:], preferred_element_type=f32)
    val = meanoff_ref[:, 0:1] + offv[:, 0:1]
    maskf = (site_ref[:] != 0).astype(f32)
    col = jax.lax.broadcasted_iota(jnp.int32, (_B, _NA), 1)
    row = jax.lax.broadcasted_iota(jnp.int32, (_B, _NA), 0)
    boh = (col // _N == row).astype(f32)
    sums = jnp.dot(boh, val * maskf, preferred_element_type=f32)
    cnts = jnp.dot(boh, maskf, preferred_element_type=f32)
    out_ref[:] = sums / jnp.maximum(cnts, 1.0)


@jax.jit
def _tc_readout(atom, meanoff, site_flat, offW):
    return pl.pallas_call(
        _readout_body,
        in_specs=[
            pl.BlockSpec((_NA, _EMBED), lambda: (0, 0)),
            pl.BlockSpec((_NA, 16), lambda: (0, 0)),
            pl.BlockSpec((_NA, 1), lambda: (0, 0)),
            pl.BlockSpec((_EMBED, 128), lambda: (0, 0)),
        ],
        out_specs=pl.BlockSpec((_B, 1), lambda: (0, 0)),
        out_shape=jax.ShapeDtypeStruct((_B, 1), jnp.float32),
    )(atom, meanoff, site_flat, offW)


def _pack_jnp(x):
    bits = jax.lax.bitcast_convert_type(x, jnp.int32)
    rnd = jnp.bitwise_and(bits + jnp.int32(0x8000), jnp.int32(-65536))
    return jnp.bitwise_or(rnd[:, :128],
                          jax.lax.shift_right_logical(rnd[:, 128:], 16))


def _prep_params(p):
    emb = jnp.zeros((128, 384), jnp.float32)
    emb = emb.at[:_MAX_ATOMIC_NUM, :_EMBED].set(p['atom_embedding'])
    emb = emb.at[:_MAX_ATOMIC_NUM, _EMBED:_EMBED + 1].set(p['atom_mean'])
    emb = emb.at[:, _EMBED].add(p['off_b'][0])
    cent = jnp.broadcast_to(p['rbf_centers'][None, :], (8, _RBF_DIM))
    gap = jnp.full((8, 128), p['rbf_gap'], jnp.float32)
    bondb = jnp.broadcast_to(p['bond_b'][None, :], (8, _EMBED))
    offW = jnp.zeros((_EMBED, 128), jnp.float32)
    offW = offW.at[:, 0:1].set(p['off_W'])
    return emb, cent, gap, bondb, offW


def _gnn_pass(site_flat, site_i, dist, dstT, idx_cat, params):
    emb, cent, gap, bondb, offW = _prep_params(params)
    dist_flat = dist.reshape(_NE, 1)
    b8 = lambda b: jnp.broadcast_to(b[None, :], (8, b.shape[0]))

    cat = _sc_gather(emb, site_i, 384, _NA)
    atom = cat[:, :_EMBED]
    meanoff = cat[:, _EMBED:_EMBED + 16]
    atom_pk = _pack_jnp(atom)

    bond = _tc_bond_init(dist_flat, cent, gap, params['bond_W'], bondb)

    for i in range(_NUM_MESSAGES):
        ep = params['edge'][i]
        npar = params['node'][i]
        gath = _sc_gather(atom_pk, idx_cat, 128, 2 * _NE,
                          dtype=jnp.int32)
        bond, agg = _tc_step(
            bond, gath, dist_flat, dstT,
            ep['W1'][:_EMBED], ep['W1'][_EMBED:].astype(jnp.bfloat16),
            b8(ep['b1']), ep['W2'], b8(ep['b2']),
            npar['W1'][:_EMBED].astype(jnp.bfloat16), npar['W1'][_EMBED:],
            b8(npar['b1']), npar['W2'], b8(npar['b2']))
        atom, atom_pk = _tc_node(agg, atom, npar['W3'], b8(npar['b3']),
                                 npar['W4'], b8(npar['b4']))

    return _tc_readout(atom, meanoff, site_flat, offW)


def kernel(site, distance, connectivity, input_vol, true_vol, vol_params,
           energy_params):
    site = site.astype(jnp.int32)
    conn = connectivity.astype(jnp.int32)
    site_flat = site.reshape(_NA, 1)
    site_i = site.reshape(_NA // 32, 32)
    offs = (jnp.arange(_B, dtype=jnp.int32) * _N)[:, None]
    dst_g = (conn[:, :, 0] + offs).reshape(_NE)
    src_g = (conn[:, :, 1] + offs).reshape(_NE)
    idx_cat = jnp.concatenate([src_g, dst_g]).reshape(2 * _NE // 128, 128)
    dstT = dst_g.reshape(_NBLK, 1, _EBLK)

    pred_vol = _gnn_pass(site_flat, site_i, distance, dstT, idx_cat,
                         vol_params)
    dist2 = distance * jnp.power(pred_vol / input_vol, 1.0 / 3.0)
    pred_energy = _gnn_pass(site_flat, site_i, dist2, dstT, idx_cat,
                            energy_params)
    return pred_vol, pred_energy


# speedup vs baseline: 1.0828x; 1.0828x over previous
import functools

import jax
import jax.numpy as jnp
from jax import lax
from jax.experimental import pallas as pl
from jax.experimental.pallas import tpu as pltpu
from jax.experimental.pallas import tpu_sc as plsc

_MAX_ATOMIC_NUM = 84
_EMBED = 256
_NUM_MESSAGES = 6
_RBF_DIM = 128
_B, _N, _E = 8, 128, 1024
_NA = _B * _N
_NE = _B * _E
_EBLK = 512
_NBLK = _NE // _EBLK

_NC, _NS = 2, 16
_NW = _NC * _NS


def _sc_gather(table, idx2, D, B, dtype=jnp.float32):
    b_per_w = B // _NW
    ch = idx2.shape[1]
    n_ch = b_per_w // ch
    mesh = plsc.VectorSubcoreMesh(core_axis_name="c", subcore_axis_name="s")

    @functools.partial(
        pl.kernel, mesh=mesh,
        out_type=jax.ShapeDtypeStruct((B, D), dtype),
        scratch_types=[
            pltpu.VMEM((n_ch, ch), jnp.int32),
            pltpu.VMEM((2, ch, D), dtype),
            pltpu.SemaphoreType.DMA,
            pltpu.SemaphoreType.DMA,
        ],
    )
    def k(table_hbm, idx_hbm, out_hbm, idx_v, rows_v, gsem, ssem):
        wid = lax.axis_index("s") * _NC + lax.axis_index("c")
        base = wid * b_per_w
        pltpu.sync_copy(idx_hbm.at[pl.ds(wid * n_ch, n_ch)], idx_v)
        gathers = [None] * n_ch
        stores = [None] * n_ch
        gathers[0] = pltpu.async_copy(table_hbm.at[idx_v.at[0]],
                                      rows_v.at[0], gsem)
        for j in range(n_ch):
            b = j % 2
            if j + 1 < n_ch:
                if j - 1 >= 0:
                    stores[j - 1].wait()
                gathers[j + 1] = pltpu.async_copy(
                    table_hbm.at[idx_v.at[j + 1]], rows_v.at[1 - b], gsem)
            gathers[j].wait()
            stores[j] = pltpu.async_copy(
                rows_v.at[b], out_hbm.at[pl.ds(base + j * ch, ch)], ssem)
        for j in range(max(0, n_ch - 2), n_ch):
            stores[j].wait()

    return k(table, idx2)


def _bond_init_body(dist_ref, cent_ref, gap_ref, bondW_ref, bondb_ref,
                    out_ref):
    f32 = jnp.float32
    d = dist_ref[:]
    d = jnp.where(d != d, jnp.zeros_like(d), d)
    off = d - cent_ref[0:1, :]
    rbf = jnp.exp(-gap_ref[0:1, :] * off * off)
    b0 = jnp.dot(rbf, bondW_ref[:], preferred_element_type=f32)
    out_ref[:] = b0 + bondb_ref[0:1, :]


@jax.jit
def _tc_bond_init(dist_flat, cent, gap, bondW, bondb):
    eblk = pl.BlockSpec((_EBLK, 1), lambda b: (b, 0))
    return pl.pallas_call(
        _bond_init_body,
        grid=(_NBLK,),
        in_specs=[
            eblk,
            pl.BlockSpec((8, 128), lambda b: (0, 0)),
            pl.BlockSpec((8, 128), lambda b: (0, 0)),
            pl.BlockSpec((_RBF_DIM, _EMBED), lambda b: (0, 0)),
            pl.BlockSpec((8, _EMBED), lambda b: (0, 0)),
        ],
        out_specs=pl.BlockSpec((_EBLK, _EMBED), lambda b: (b, 0)),
        out_shape=jax.ShapeDtypeStruct((_NE, _EMBED), jnp.float32),
    )(dist_flat, cent, gap, bondW, bondb)


def _unpack(p):
    f32 = jnp.float32
    hi = jax.lax.bitcast_convert_type(
        jnp.bitwise_and(p, jnp.int32(-65536)), f32)
    lo = jax.lax.bitcast_convert_type(jnp.left_shift(p, 16), f32)
    return jnp.concatenate([hi, lo], axis=1)


def _pack(x):
    i32 = jnp.int32
    bits = jax.lax.bitcast_convert_type(x, i32)
    rnd = jnp.bitwise_and(bits + jnp.int32(0x8000), jnp.int32(-65536))
    hi = rnd[:, :128]
    lo = jax.lax.shift_right_logical(rnd[:, 128:], 16)
    return jnp.bitwise_or(hi, lo)


def _step_body(bond_ref, src_ref, dst_ref, dist_ref, dstT_ref, atom_ref,
               eW1_ref, eb1_ref, eW2_ref, eb2_ref,
               nW1_ref, nb1_ref, nW2_ref, nb2_ref,
               nW3_ref, nb3_ref, nW4_ref, nb4_ref,
               bond_out_ref, atom_out_ref, atom_pk_out_ref, agg_s):
    f32 = jnp.float32
    blk = pl.program_id(0)
    bond = bond_ref[:]
    src_a = _unpack(src_ref[:])
    dst_a = _unpack(dst_ref[:])
    bf16 = jnp.bfloat16
    bond_bf = bond.astype(bf16)
    src_bf = src_a.astype(bf16)
    dst_bf = dst_a.astype(bf16)
    h = (jnp.dot(bond_bf, eW1_ref[:_EMBED, :], preferred_element_type=f32)
         + jnp.dot(src_bf, eW1_ref[_EMBED:2 * _EMBED, :], preferred_element_type=f32)
         + jnp.dot(dst_bf, eW1_ref[2 * _EMBED:, :], preferred_element_type=f32)
         + eb1_ref[0:1, :])
    h = jnp.maximum(h, 0.0)
    h = jnp.dot(h.astype(bf16), eW2_ref[:], preferred_element_type=f32) + eb2_ref[0:1, :]
    bond_new = bond + h
    bond_out_ref[:] = bond_new
    m = (jnp.dot(src_bf, nW1_ref[:_EMBED, :], preferred_element_type=f32)
         + jnp.dot(bond_new.astype(bf16), nW1_ref[_EMBED:, :], preferred_element_type=f32)
         + nb1_ref[0:1, :])
    m = jnp.maximum(m, 0.0)
    m = jnp.dot(m.astype(bf16), nW2_ref[:], preferred_element_type=f32) + nb2_ref[0:1, :]
    dmask = (dist_ref[:] != 0.0).astype(f32)
    m = m * dmask

    row_i = jax.lax.broadcasted_iota(jnp.int32, (_NA, _EBLK), 0)
    ohT = (row_i == dstT_ref[0]).astype(jnp.bfloat16)
    contrib = jnp.dot(ohT, m.astype(jnp.bfloat16), preferred_element_type=f32)

    @pl.when(blk == 0)
    def _set():
        agg_s[:] = contrib

    @pl.when(blk != 0)
    def _add():
        agg_s[:] = agg_s[:] + contrib

    @pl.when(blk == _NBLK - 1)
    def _node():
        a = jnp.dot(agg_s[:].astype(jnp.bfloat16), nW3_ref[:],
                    preferred_element_type=f32) + nb3_ref[0:1, :]
        a = jnp.maximum(a, 0.0)
        a = jnp.dot(a.astype(jnp.bfloat16), nW4_ref[:],
                    preferred_element_type=f32) + nb4_ref[0:1, :]
        new_atom = atom_ref[:] + a
        atom_out_ref[:] = new_atom
        atom_pk_out_ref[:] = _pack(new_atom)


@jax.jit
def _tc_step(bond, gath, dist_flat, dstT, atom,
             eW1, eb1, eW2, eb2, nW1, nb1, nW2, nb2, nW3, nb3, nW4, nb4):
    eblk = pl.BlockSpec((_EBLK, _EMBED), lambda b: (b, 0))
    c2 = lambda shp: pl.BlockSpec(shp, lambda b: (0, 0))
    return pl.pallas_call(
        _step_body,
        grid=(_NBLK,),
        in_specs=[
            eblk,
            pl.BlockSpec((_EBLK, 128), lambda b: (b, 0)),
            pl.BlockSpec((_EBLK, 128), lambda b: (b + _NBLK, 0)),
            pl.BlockSpec((_EBLK, 1), lambda b: (b, 0)),
            pl.BlockSpec((1, 1, _EBLK), lambda b: (b, 0, 0)),
            c2((_NA, _EMBED)),
            c2((3 * _EMBED, 2 * _EMBED)), c2((8, 2 * _EMBED)),
            c2((2 * _EMBED, _EMBED)), c2((8, _EMBED)),
            c2((2 * _EMBED, 2 * _EMBED)), c2((8, 2 * _EMBED)),
            c2((2 * _EMBED, _EMBED)), c2((8, _EMBED)),
            c2((_EMBED, 2 * _EMBED)), c2((8, 2 * _EMBED)),
            c2((2 * _EMBED, _EMBED)), c2((8, _EMBED)),
        ],
        out_specs=[eblk, pl.BlockSpec((_NA, _EMBED), lambda b: (0, 0)),
                   pl.BlockSpec((_NA, 128), lambda b: (0, 0))],
        out_shape=[jax.ShapeDtypeStruct((_NE, _EMBED), jnp.float32),
                   jax.ShapeDtypeStruct((_NA, _EMBED), jnp.float32),
                   jax.ShapeDtypeStruct((_NA, 128), jnp.int32)],
        input_output_aliases={0: 0},
        scratch_shapes=[pltpu.VMEM((_NA, _EMBED), jnp.float32)],
    )(bond, gath, gath, dist_flat, dstT, atom,
      eW1, eb1, eW2, eb2, nW1, nb1, nW2, nb2, nW3, nb3, nW4, nb4)


def _readout_body(atom_ref, meanoff_ref, site_ref, offW_ref, out_ref):
    f32 = jnp.float32
    offv = jnp.dot(atom_ref[:], offW_ref[:], preferred_element_type=f32)
    val = meanoff_ref[:, 0:1] + offv[:, 0:1]
    maskf = (site_ref[:] != 0).astype(f32)
    col = jax.lax.broadcasted_iota(jnp.int32, (_B, _NA), 1)
    row = jax.lax.broadcasted_iota(jnp.int32, (_B, _NA), 0)
    boh = (col // _N == row).astype(f32)
    sums = jnp.dot(boh, val * maskf, preferred_element_type=f32)
    cnts = jnp.dot(boh, maskf, preferred_element_type=f32)
    out_ref[:] = sums / jnp.maximum(cnts, 1.0)


@jax.jit
def _tc_readout(atom, meanoff, site_flat, offW):
    return pl.pallas_call(
        _readout_body,
        in_specs=[
            pl.BlockSpec((_NA, _EMBED), lambda: (0, 0)),
            pl.BlockSpec((_NA, 16), lambda: (0, 0)),
            pl.BlockSpec((_NA, 1), lambda: (0, 0)),
            pl.BlockSpec((_EMBED, 128), lambda: (0, 0)),
        ],
        out_specs=pl.BlockSpec((_B, 1), lambda: (0, 0)),
        out_shape=jax.ShapeDtypeStruct((_B, 1), jnp.float32),
    )(atom, meanoff, site_flat, offW)


def _pack_jnp(x):
    bits = jax.lax.bitcast_convert_type(x, jnp.int32)
    rnd = jnp.bitwise_and(bits + jnp.int32(0x8000), jnp.int32(-65536))
    return jnp.bitwise_or(rnd[:, :128],
                          jax.lax.shift_right_logical(rnd[:, 128:], 16))


def _prep_params(p):
    emb = jnp.zeros((128, 384), jnp.float32)
    emb = emb.at[:_MAX_ATOMIC_NUM, :_EMBED].set(p['atom_embedding'])
    emb = emb.at[:_MAX_ATOMIC_NUM, _EMBED:_EMBED + 1].set(p['atom_mean'])
    emb = emb.at[:, _EMBED].add(p['off_b'][0])
    cent = jnp.broadcast_to(p['rbf_centers'][None, :], (8, _RBF_DIM))
    gap = jnp.full((8, 128), p['rbf_gap'], jnp.float32)
    bondb = jnp.broadcast_to(p['bond_b'][None, :], (8, _EMBED))
    offW = jnp.zeros((_EMBED, 128), jnp.float32)
    offW = offW.at[:, 0:1].set(p['off_W'])
    return emb, cent, gap, bondb, offW


def _gnn_pass(site_flat, site_i, dist, dstT, idx_cat, params):
    emb, cent, gap, bondb, offW = _prep_params(params)
    dist_flat = dist.reshape(_NE, 1)
    b8 = lambda b: jnp.broadcast_to(b[None, :], (8, b.shape[0]))

    cat = _sc_gather(emb, site_i, 384, _NA)
    atom = cat[:, :_EMBED]
    meanoff = cat[:, _EMBED:_EMBED + 16]
    atom_pk = _pack_jnp(atom)

    bond = _tc_bond_init(dist_flat, cent, gap, params['bond_W'], bondb)

    for i in range(_NUM_MESSAGES):
        ep = params['edge'][i]
        npar = params['node'][i]
        gath = _sc_gather(atom_pk, idx_cat, 128, 2 * _NE,
                          dtype=jnp.int32)
        bond, atom, atom_pk = _tc_step(
            bond, gath, dist_flat, dstT, atom,
            ep['W1'].astype(jnp.bfloat16), b8(ep['b1']),
            ep['W2'].astype(jnp.bfloat16), b8(ep['b2']),
            npar['W1'].astype(jnp.bfloat16), b8(npar['b1']),
            npar['W2'].astype(jnp.bfloat16), b8(npar['b2']),
            npar['W3'].astype(jnp.bfloat16), b8(npar['b3']),
            npar['W4'].astype(jnp.bfloat16), b8(npar['b4']))

    return _tc_readout(atom, meanoff, site_flat, offW)


def kernel(site, distance, connectivity, input_vol, true_vol, vol_params,
           energy_params):
    site = site.astype(jnp.int32)
    conn = connectivity.astype(jnp.int32)
    site_flat = site.reshape(_NA, 1)
    site_i = site.reshape(_NA // 32, 32)
    offs = (jnp.arange(_B, dtype=jnp.int32) * _N)[:, None]
    dst_g = (conn[:, :, 0] + offs).reshape(_NE)
    src_g = (conn[:, :, 1] + offs).reshape(_NE)
    idx_cat = jnp.concatenate([src_g, dst_g]).reshape(2 * _NE // 128, 128)
    dstT = dst_g.reshape(_NBLK, 1, _EBLK)

    pred_vol = _gnn_pass(site_flat, site_i, distance, dstT, idx_cat,
                         vol_params)
    dist2 = distance * jnp.power(pred_vol / input_vol, 1.0 / 3.0)
    pred_energy = _gnn_pass(site_flat, site_i, dist2, dstT, idx_cat,
                            energy_params)
    return pred_vol, pred_energy


# speedup vs baseline: 1.1696x; 1.0801x over previous
import functools

import jax
import jax.numpy as jnp
from jax import lax
from jax.experimental import pallas as pl
from jax.experimental.pallas import tpu as pltpu
from jax.experimental.pallas import tpu_sc as plsc

_MAX_ATOMIC_NUM = 84
_EMBED = 256
_NUM_MESSAGES = 6
_RBF_DIM = 128
_B, _N, _E = 8, 128, 1024
_NA = _B * _N
_NE = _B * _E
_EBLK = 1024
_NBLK = _NE // _EBLK

_NC, _NS = 2, 16
_NW = _NC * _NS


def _sc_gather(table, idx2, D, B, dtype=jnp.float32):
    b_per_w = B // _NW
    ch = idx2.shape[1]
    n_ch = b_per_w // ch
    mesh = plsc.VectorSubcoreMesh(core_axis_name="c", subcore_axis_name="s")

    @functools.partial(
        pl.kernel, mesh=mesh,
        out_type=jax.ShapeDtypeStruct((B, D), dtype),
        scratch_types=[
            pltpu.VMEM((n_ch, ch), jnp.int32),
            pltpu.VMEM((2, ch, D), dtype),
            pltpu.SemaphoreType.DMA,
            pltpu.SemaphoreType.DMA,
        ],
    )
    def k(table_hbm, idx_hbm, out_hbm, idx_v, rows_v, gsem, ssem):
        wid = lax.axis_index("s") * _NC + lax.axis_index("c")
        base = wid * b_per_w
        pltpu.sync_copy(idx_hbm.at[pl.ds(wid * n_ch, n_ch)], idx_v)
        gathers = [None] * n_ch
        stores = [None] * n_ch
        gathers[0] = pltpu.async_copy(table_hbm.at[idx_v.at[0]],
                                      rows_v.at[0], gsem)
        for j in range(n_ch):
            b = j % 2
            if j + 1 < n_ch:
                if j - 1 >= 0:
                    stores[j - 1].wait()
                gathers[j + 1] = pltpu.async_copy(
                    table_hbm.at[idx_v.at[j + 1]], rows_v.at[1 - b], gsem)
            gathers[j].wait()
            stores[j] = pltpu.async_copy(
                rows_v.at[b], out_hbm.at[pl.ds(base + j * ch, ch)], ssem)
        for j in range(max(0, n_ch - 2), n_ch):
            stores[j].wait()

    return k(table, idx2)


def _bond_init_body(dist_ref, cent_ref, gap_ref, bondW_ref, bondb_ref,
                    out_ref):
    f32 = jnp.float32
    d = dist_ref[:]
    d = jnp.where(d != d, jnp.zeros_like(d), d)
    off = d - cent_ref[0:1, :]
    rbf = jnp.exp(-gap_ref[0:1, :] * off * off)
    b0 = jnp.dot(rbf, bondW_ref[:], preferred_element_type=f32)
    out_ref[:] = b0 + bondb_ref[0:1, :]


@jax.jit
def _tc_bond_init(dist_flat, cent, gap, bondW, bondb):
    eblk = pl.BlockSpec((_EBLK, 1), lambda b: (b, 0))
    return pl.pallas_call(
        _bond_init_body,
        grid=(_NBLK,),
        in_specs=[
            eblk,
            pl.BlockSpec((8, 128), lambda b: (0, 0)),
            pl.BlockSpec((8, 128), lambda b: (0, 0)),
            pl.BlockSpec((_RBF_DIM, _EMBED), lambda b: (0, 0)),
            pl.BlockSpec((8, _EMBED), lambda b: (0, 0)),
        ],
        out_specs=pl.BlockSpec((_EBLK, _EMBED), lambda b: (b, 0)),
        out_shape=jax.ShapeDtypeStruct((_NE, _EMBED), jnp.float32),
    )(dist_flat, cent, gap, bondW, bondb)


def _unpack(p):
    f32 = jnp.float32
    hi = jax.lax.bitcast_convert_type(
        jnp.bitwise_and(p, jnp.int32(-65536)), f32)
    lo = jax.lax.bitcast_convert_type(jnp.left_shift(p, 16), f32)
    return jnp.concatenate([hi, lo], axis=1)


def _pack(x):
    i32 = jnp.int32
    bits = jax.lax.bitcast_convert_type(x, i32)
    rnd = jnp.bitwise_and(bits + jnp.int32(0x8000), jnp.int32(-65536))
    hi = rnd[:, :128]
    lo = jax.lax.shift_right_logical(rnd[:, 128:], 16)
    return jnp.bitwise_or(hi, lo)


def _step_body(bond_ref, src_ref, dst_ref, dist_ref, dstT_ref, atom_ref,
               eW1_ref, eb1_ref, eW2_ref, eb2_ref,
               nW1_ref, nb1_ref, nW2_ref, nb2_ref,
               nW3_ref, nb3_ref, nW4_ref, nb4_ref,
               bond_out_ref, atom_out_ref, atom_pk_out_ref, agg_s):
    f32 = jnp.float32
    blk = pl.program_id(0)
    bond = bond_ref[:]
    src_a = _unpack(src_ref[:])
    dst_a = _unpack(dst_ref[:])
    h = (jnp.dot(bond, eW1_ref[:_EMBED, :], preferred_element_type=f32)
         + jnp.dot(src_a, eW1_ref[_EMBED:2 * _EMBED, :], preferred_element_type=f32)
         + jnp.dot(dst_a, eW1_ref[2 * _EMBED:, :], preferred_element_type=f32)
         + eb1_ref[0:1, :])
    h = jnp.maximum(h, 0.0)
    h = jnp.dot(h, eW2_ref[:], preferred_element_type=f32) + eb2_ref[0:1, :]
    bond_new = bond + h
    bond_out_ref[:] = bond_new
    m = (jnp.dot(src_a, nW1_ref[:_EMBED, :], preferred_element_type=f32)
         + jnp.dot(bond_new, nW1_ref[_EMBED:, :], preferred_element_type=f32)
         + nb1_ref[0:1, :])
    m = jnp.maximum(m, 0.0)
    m = jnp.dot(m, nW2_ref[:], preferred_element_type=f32) + nb2_ref[0:1, :]
    dmask = (dist_ref[:] != 0.0).astype(f32)
    m = m * dmask

    row_i = jax.lax.broadcasted_iota(jnp.int32, (_NA, _EBLK), 0)
    ohT = (row_i == dstT_ref[0]).astype(jnp.bfloat16)
    contrib = jnp.dot(ohT, m.astype(jnp.bfloat16), preferred_element_type=f32)

    @pl.when(blk == 0)
    def _set():
        agg_s[:] = contrib

    @pl.when(blk != 0)
    def _add():
        agg_s[:] = agg_s[:] + contrib

    @pl.when(blk == _NBLK - 1)
    def _node():
        a = jnp.dot(agg_s[:], nW3_ref[:], preferred_element_type=f32) + nb3_ref[0:1, :]
        a = jnp.maximum(a, 0.0)
        a = jnp.dot(a, nW4_ref[:], preferred_element_type=f32) + nb4_ref[0:1, :]
        new_atom = atom_ref[:] + a
        atom_out_ref[:] = new_atom
        atom_pk_out_ref[:] = _pack(new_atom)


@jax.jit
def _tc_step(bond, gath, dist_flat, dstT, atom,
             eW1, eb1, eW2, eb2, nW1, nb1, nW2, nb2, nW3, nb3, nW4, nb4):
    eblk = pl.BlockSpec((_EBLK, _EMBED), lambda b: (b, 0))
    c2 = lambda shp: pl.BlockSpec(shp, lambda b: (0, 0))
    return pl.pallas_call(
        _step_body,
        grid=(_NBLK,),
        in_specs=[
            eblk,
            pl.BlockSpec((_EBLK, 128), lambda b: (b, 0)),
            pl.BlockSpec((_EBLK, 128), lambda b: (b + _NBLK, 0)),
            pl.BlockSpec((_EBLK, 1), lambda b: (b, 0)),
            pl.BlockSpec((1, 1, _EBLK), lambda b: (b, 0, 0)),
            c2((_NA, _EMBED)),
            c2((3 * _EMBED, 2 * _EMBED)), c2((8, 2 * _EMBED)),
            c2((2 * _EMBED, _EMBED)), c2((8, _EMBED)),
            c2((2 * _EMBED, 2 * _EMBED)), c2((8, 2 * _EMBED)),
            c2((2 * _EMBED, _EMBED)), c2((8, _EMBED)),
            c2((_EMBED, 2 * _EMBED)), c2((8, 2 * _EMBED)),
            c2((2 * _EMBED, _EMBED)), c2((8, _EMBED)),
        ],
        out_specs=[eblk, pl.BlockSpec((_NA, _EMBED), lambda b: (0, 0)),
                   pl.BlockSpec((_NA, 128), lambda b: (0, 0))],
        out_shape=[jax.ShapeDtypeStruct((_NE, _EMBED), jnp.float32),
                   jax.ShapeDtypeStruct((_NA, _EMBED), jnp.float32),
                   jax.ShapeDtypeStruct((_NA, 128), jnp.int32)],
        input_output_aliases={0: 0},
        scratch_shapes=[pltpu.VMEM((_NA, _EMBED), jnp.float32)],
    )(bond, gath, gath, dist_flat, dstT, atom,
      eW1, eb1, eW2, eb2, nW1, nb1, nW2, nb2, nW3, nb3, nW4, nb4)


def _readout_body(atom_ref, meanoff_ref, site_ref, offW_ref, out_ref):
    f32 = jnp.float32
    offv = jnp.dot(atom_ref[:], offW_ref[:], preferred_element_type=f32)
    val = meanoff_ref[:, 0:1] + offv[:, 0:1]
    maskf = (site_ref[:] != 0).astype(f32)
    col = jax.lax.broadcasted_iota(jnp.int32, (_B, _NA), 1)
    row = jax.lax.broadcasted_iota(jnp.int32, (_B, _NA), 0)
    boh = (col // _N == row).astype(f32)
    sums = jnp.dot(boh, val * maskf, preferred_element_type=f32)
    cnts = jnp.dot(boh, maskf, preferred_element_type=f32)
    out_ref[:] = sums / jnp.maximum(cnts, 1.0)


@jax.jit
def _tc_readout(atom, meanoff, site_flat, offW):
    return pl.pallas_call(
        _readout_body,
        in_specs=[
            pl.BlockSpec((_NA, _EMBED), lambda: (0, 0)),
            pl.BlockSpec((_NA, 16), lambda: (0, 0)),
            pl.BlockSpec((_NA, 1), lambda: (0, 0)),
            pl.BlockSpec((_EMBED, 128), lambda: (0, 0)),
        ],
        out_specs=pl.BlockSpec((_B, 1), lambda: (0, 0)),
        out_shape=jax.ShapeDtypeStruct((_B, 1), jnp.float32),
    )(atom, meanoff, site_flat, offW)


def _pack_jnp(x):
    bits = jax.lax.bitcast_convert_type(x, jnp.int32)
    rnd = jnp.bitwise_and(bits + jnp.int32(0x8000), jnp.int32(-65536))
    return jnp.bitwise_or(rnd[:, :128],
                          jax.lax.shift_right_logical(rnd[:, 128:], 16))


def _prep_params(p):
    emb = jnp.zeros((128, 384), jnp.float32)
    emb = emb.at[:_MAX_ATOMIC_NUM, :_EMBED].set(p['atom_embedding'])
    emb = emb.at[:_MAX_ATOMIC_NUM, _EMBED:_EMBED + 1].set(p['atom_mean'])
    emb = emb.at[:, _EMBED].add(p['off_b'][0])
    cent = jnp.broadcast_to(p['rbf_centers'][None, :], (8, _RBF_DIM))
    gap = jnp.full((8, 128), p['rbf_gap'], jnp.float32)
    bondb = jnp.broadcast_to(p['bond_b'][None, :], (8, _EMBED))
    offW = jnp.zeros((_EMBED, 128), jnp.float32)
    offW = offW.at[:, 0:1].set(p['off_W'])
    return emb, cent, gap, bondb, offW


def _gnn_pass(site_flat, site_i, dist, dstT, idx_cat, params):
    emb, cent, gap, bondb, offW = _prep_params(params)
    dist_flat = dist.reshape(_NE, 1)
    b8 = lambda b: jnp.broadcast_to(b[None, :], (8, b.shape[0]))

    cat = _sc_gather(emb, site_i, 384, _NA)
    atom = cat[:, :_EMBED]
    meanoff = cat[:, _EMBED:_EMBED + 16]
    atom_pk = _pack_jnp(atom)

    bond = _tc_bond_init(dist_flat, cent, gap, params['bond_W'], bondb)

    for i in range(_NUM_MESSAGES):
        ep = params['edge'][i]
        npar = params['node'][i]
        gath = _sc_gather(atom_pk, idx_cat, 128, 2 * _NE,
                          dtype=jnp.int32)
        bond, atom, atom_pk = _tc_step(
            bond, gath, dist_flat, dstT, atom,
            ep['W1'], b8(ep['b1']), ep['W2'], b8(ep['b2']),
            npar['W1'], b8(npar['b1']), npar['W2'], b8(npar['b2']),
            npar['W3'], b8(npar['b3']), npar['W4'], b8(npar['b4']))

    return _tc_readout(atom, meanoff, site_flat, offW)


def kernel(site, distance, connectivity, input_vol, true_vol, vol_params,
           energy_params):
    site = site.astype(jnp.int32)
    conn = connectivity.astype(jnp.int32)
    site_flat = site.reshape(_NA, 1)
    site_i = site.reshape(_NA // 32, 32)
    offs = (jnp.arange(_B, dtype=jnp.int32) * _N)[:, None]
    dst_g = (conn[:, :, 0] + offs).reshape(_NE)
    src_g = (conn[:, :, 1] + offs).reshape(_NE)
    idx_cat = jnp.concatenate([src_g, dst_g]).reshape(2 * _NE // 128, 128)
    dstT = dst_g.reshape(_NBLK, 1, _EBLK)

    pred_vol = _gnn_pass(site_flat, site_i, distance, dstT, idx_cat,
                         vol_params)
    dist2 = distance * jnp.power(pred_vol / input_vol, 1.0 / 3.0)
    pred_energy = _gnn_pass(site_flat, site_i, dist2, dstT, idx_cat,
                            energy_params)
    return pred_vol, pred_energy


# speedup vs baseline: 1.1933x; 1.0203x over previous
import functools

import jax
import jax.numpy as jnp
from jax import lax
from jax.experimental import pallas as pl
from jax.experimental.pallas import tpu as pltpu
from jax.experimental.pallas import tpu_sc as plsc

_MAX_ATOMIC_NUM = 84
_EMBED = 256
_NUM_MESSAGES = 6
_RBF_DIM = 128
_B, _N, _E = 8, 128, 1024
_NA = _B * _N
_NE = _B * _E
_EBLK = 2048
_NBLK = _NE // _EBLK

_NC, _NS = 2, 16
_NW = _NC * _NS


def _sc_gather(table, idx2, D, B, dtype=jnp.float32):
    b_per_w = B // _NW
    ch = idx2.shape[1]
    n_ch = b_per_w // ch
    mesh = plsc.VectorSubcoreMesh(core_axis_name="c", subcore_axis_name="s")

    @functools.partial(
        pl.kernel, mesh=mesh,
        out_type=jax.ShapeDtypeStruct((B, D), dtype),
        scratch_types=[
            pltpu.VMEM((n_ch, ch), jnp.int32),
            pltpu.VMEM((2, ch, D), dtype),
            pltpu.SemaphoreType.DMA,
            pltpu.SemaphoreType.DMA,
        ],
    )
    def k(table_hbm, idx_hbm, out_hbm, idx_v, rows_v, gsem, ssem):
        wid = lax.axis_index("s") * _NC + lax.axis_index("c")
        base = wid * b_per_w
        pltpu.sync_copy(idx_hbm.at[pl.ds(wid * n_ch, n_ch)], idx_v)
        gathers = [None] * n_ch
        stores = [None] * n_ch
        gathers[0] = pltpu.async_copy(table_hbm.at[idx_v.at[0]],
                                      rows_v.at[0], gsem)
        for j in range(n_ch):
            b = j % 2
            if j + 1 < n_ch:
                if j - 1 >= 0:
                    stores[j - 1].wait()
                gathers[j + 1] = pltpu.async_copy(
                    table_hbm.at[idx_v.at[j + 1]], rows_v.at[1 - b], gsem)
            gathers[j].wait()
            stores[j] = pltpu.async_copy(
                rows_v.at[b], out_hbm.at[pl.ds(base + j * ch, ch)], ssem)
        for j in range(max(0, n_ch - 2), n_ch):
            stores[j].wait()

    return k(table, idx2)


def _bond_init_body(dist_ref, cent_ref, gap_ref, bondW_ref, bondb_ref,
                    out_ref):
    f32 = jnp.float32
    d = dist_ref[:]
    d = jnp.where(d != d, jnp.zeros_like(d), d)
    off = d - cent_ref[0:1, :]
    rbf = jnp.exp(-gap_ref[0:1, :] * off * off)
    b0 = jnp.dot(rbf, bondW_ref[:], preferred_element_type=f32)
    out_ref[:] = b0 + bondb_ref[0:1, :]


@jax.jit
def _tc_bond_init(dist_flat, cent, gap, bondW, bondb):
    eblk = pl.BlockSpec((_EBLK, 1), lambda b: (b, 0))
    return pl.pallas_call(
        _bond_init_body,
        grid=(_NBLK,),
        in_specs=[
            eblk,
            pl.BlockSpec((8, 128), lambda b: (0, 0)),
            pl.BlockSpec((8, 128), lambda b: (0, 0)),
            pl.BlockSpec((_RBF_DIM, _EMBED), lambda b: (0, 0)),
            pl.BlockSpec((8, _EMBED), lambda b: (0, 0)),
        ],
        out_specs=pl.BlockSpec((_EBLK, _EMBED), lambda b: (b, 0)),
        out_shape=jax.ShapeDtypeStruct((_NE, _EMBED), jnp.float32),
    )(dist_flat, cent, gap, bondW, bondb)


def _unpack(p):
    f32 = jnp.float32
    hi = jax.lax.bitcast_convert_type(
        jnp.bitwise_and(p, jnp.int32(-65536)), f32)
    lo = jax.lax.bitcast_convert_type(jnp.left_shift(p, 16), f32)
    return jnp.concatenate([hi, lo], axis=1)


def _pack(x):
    i32 = jnp.int32
    bits = jax.lax.bitcast_convert_type(x, i32)
    rnd = jnp.bitwise_and(bits + jnp.int32(0x8000), jnp.int32(-65536))
    hi = rnd[:, :128]
    lo = jax.lax.shift_right_logical(rnd[:, 128:], 16)
    return jnp.bitwise_or(hi, lo)


def _step_body(bond_ref, src_ref, dst_ref, dist_ref, dstT_ref, atom_ref,
               eW1_ref, eb1_ref, eW2_ref, eb2_ref,
               nW1_ref, nb1_ref, nW2_ref, nb2_ref,
               nW3_ref, nb3_ref, nW4_ref, nb4_ref,
               bond_out_ref, atom_out_ref, atom_pk_out_ref, agg_s):
    f32 = jnp.float32
    blk = pl.program_id(0)
    bond = bond_ref[:]
    src_a = _unpack(src_ref[:])
    dst_a = _unpack(dst_ref[:])
    h = (jnp.dot(bond, eW1_ref[:_EMBED, :], preferred_element_type=f32)
         + jnp.dot(src_a, eW1_ref[_EMBED:2 * _EMBED, :], preferred_element_type=f32)
         + jnp.dot(dst_a, eW1_ref[2 * _EMBED:, :], preferred_element_type=f32)
         + eb1_ref[0:1, :])
    h = jnp.maximum(h, 0.0)
    h = jnp.dot(h, eW2_ref[:], preferred_element_type=f32) + eb2_ref[0:1, :]
    bond_new = bond + h
    bond_out_ref[:] = bond_new
    m = (jnp.dot(src_a, nW1_ref[:_EMBED, :], preferred_element_type=f32)
         + jnp.dot(bond_new, nW1_ref[_EMBED:, :], preferred_element_type=f32)
         + nb1_ref[0:1, :])
    m = jnp.maximum(m, 0.0)
    m = jnp.dot(m, nW2_ref[:], preferred_element_type=f32) + nb2_ref[0:1, :]
    dmask = (dist_ref[:] != 0.0).astype(f32)
    m = m * dmask

    row_i = jax.lax.broadcasted_iota(jnp.int32, (_NA, _EBLK), 0)
    ohT = (row_i == dstT_ref[0]).astype(jnp.bfloat16)
    contrib = jnp.dot(ohT, m.astype(jnp.bfloat16), preferred_element_type=f32)

    @pl.when(blk == 0)
    def _set():
        agg_s[:] = contrib

    @pl.when(blk != 0)
    def _add():
        agg_s[:] = agg_s[:] + contrib

    @pl.when(blk == _NBLK - 1)
    def _node():
        a = jnp.dot(agg_s[:], nW3_ref[:], preferred_element_type=f32) + nb3_ref[0:1, :]
        a = jnp.maximum(a, 0.0)
        a = jnp.dot(a, nW4_ref[:], preferred_element_type=f32) + nb4_ref[0:1, :]
        new_atom = atom_ref[:] + a
        atom_out_ref[:] = new_atom
        atom_pk_out_ref[:] = _pack(new_atom)


@jax.jit
def _tc_step(bond, gath, dist_flat, dstT, atom,
             eW1, eb1, eW2, eb2, nW1, nb1, nW2, nb2, nW3, nb3, nW4, nb4):
    eblk = pl.BlockSpec((_EBLK, _EMBED), lambda b: (b, 0))
    c2 = lambda shp: pl.BlockSpec(shp, lambda b: (0, 0))
    return pl.pallas_call(
        _step_body,
        grid=(_NBLK,),
        in_specs=[
            eblk,
            pl.BlockSpec((_EBLK, 128), lambda b: (b, 0)),
            pl.BlockSpec((_EBLK, 128), lambda b: (b + _NBLK, 0)),
            pl.BlockSpec((_EBLK, 1), lambda b: (b, 0)),
            pl.BlockSpec((1, 1, _EBLK), lambda b: (b, 0, 0)),
            c2((_NA, _EMBED)),
            c2((3 * _EMBED, 2 * _EMBED)), c2((8, 2 * _EMBED)),
            c2((2 * _EMBED, _EMBED)), c2((8, _EMBED)),
            c2((2 * _EMBED, 2 * _EMBED)), c2((8, 2 * _EMBED)),
            c2((2 * _EMBED, _EMBED)), c2((8, _EMBED)),
            c2((_EMBED, 2 * _EMBED)), c2((8, 2 * _EMBED)),
            c2((2 * _EMBED, _EMBED)), c2((8, _EMBED)),
        ],
        out_specs=[eblk, pl.BlockSpec((_NA, _EMBED), lambda b: (0, 0)),
                   pl.BlockSpec((_NA, 128), lambda b: (0, 0))],
        out_shape=[jax.ShapeDtypeStruct((_NE, _EMBED), jnp.float32),
                   jax.ShapeDtypeStruct((_NA, _EMBED), jnp.float32),
                   jax.ShapeDtypeStruct((_NA, 128), jnp.int32)],
        input_output_aliases={0: 0},
        scratch_shapes=[pltpu.VMEM((_NA, _EMBED), jnp.float32)],
    )(bond, gath, gath, dist_flat, dstT, atom,
      eW1, eb1, eW2, eb2, nW1, nb1, nW2, nb2, nW3, nb3, nW4, nb4)


def _readout_body(atom_ref, meanoff_ref, site_ref, offW_ref, out_ref):
    f32 = jnp.float32
    offv = jnp.dot(atom_ref[:], offW_ref[:], preferred_element_type=f32)
    val = meanoff_ref[:, 0:1] + offv[:, 0:1]
    maskf = (site_ref[:] != 0).astype(f32)
    col = jax.lax.broadcasted_iota(jnp.int32, (_B, _NA), 1)
    row = jax.lax.broadcasted_iota(jnp.int32, (_B, _NA), 0)
    boh = (col // _N == row).astype(f32)
    sums = jnp.dot(boh, val * maskf, preferred_element_type=f32)
    cnts = jnp.dot(boh, maskf, preferred_element_type=f32)
    out_ref[:] = sums / jnp.maximum(cnts, 1.0)


@jax.jit
def _tc_readout(atom, meanoff, site_flat, offW):
    return pl.pallas_call(
        _readout_body,
        in_specs=[
            pl.BlockSpec((_NA, _EMBED), lambda: (0, 0)),
            pl.BlockSpec((_NA, 16), lambda: (0, 0)),
            pl.BlockSpec((_NA, 1), lambda: (0, 0)),
            pl.BlockSpec((_EMBED, 128), lambda: (0, 0)),
        ],
        out_specs=pl.BlockSpec((_B, 1), lambda: (0, 0)),
        out_shape=jax.ShapeDtypeStruct((_B, 1), jnp.float32),
    )(atom, meanoff, site_flat, offW)


def _pack_jnp(x):
    bits = jax.lax.bitcast_convert_type(x, jnp.int32)
    rnd = jnp.bitwise_and(bits + jnp.int32(0x8000), jnp.int32(-65536))
    return jnp.bitwise_or(rnd[:, :128],
                          jax.lax.shift_right_logical(rnd[:, 128:], 16))


def _prep_params(p):
    emb = jnp.zeros((128, 384), jnp.float32)
    emb = emb.at[:_MAX_ATOMIC_NUM, :_EMBED].set(p['atom_embedding'])
    emb = emb.at[:_MAX_ATOMIC_NUM, _EMBED:_EMBED + 1].set(p['atom_mean'])
    emb = emb.at[:, _EMBED].add(p['off_b'][0])
    cent = jnp.broadcast_to(p['rbf_centers'][None, :], (8, _RBF_DIM))
    gap = jnp.full((8, 128), p['rbf_gap'], jnp.float32)
    bondb = jnp.broadcast_to(p['bond_b'][None, :], (8, _EMBED))
    offW = jnp.zeros((_EMBED, 128), jnp.float32)
    offW = offW.at[:, 0:1].set(p['off_W'])
    return emb, cent, gap, bondb, offW


def _gnn_pass(site_flat, site_i, dist, dstT, idx_cat, params):
    emb, cent, gap, bondb, offW = _prep_params(params)
    dist_flat = dist.reshape(_NE, 1)
    b8 = lambda b: jnp.broadcast_to(b[None, :], (8, b.shape[0]))

    cat = _sc_gather(emb, site_i, 384, _NA)
    atom = cat[:, :_EMBED]
    meanoff = cat[:, _EMBED:_EMBED + 16]
    atom_pk = _pack_jnp(atom)

    bond = _tc_bond_init(dist_flat, cent, gap, params['bond_W'], bondb)

    for i in range(_NUM_MESSAGES):
        ep = params['edge'][i]
        npar = params['node'][i]
        gath = _sc_gather(atom_pk, idx_cat, 128, 2 * _NE,
                          dtype=jnp.int32)
        bond, atom, atom_pk = _tc_step(
            bond, gath, dist_flat, dstT, atom,
            ep['W1'], b8(ep['b1']), ep['W2'], b8(ep['b2']),
            npar['W1'], b8(npar['b1']), npar['W2'], b8(npar['b2']),
            npar['W3'], b8(npar['b3']), npar['W4'], b8(npar['b4']))

    return _tc_readout(atom, meanoff, site_flat, offW)


def kernel(site, distance, connectivity, input_vol, true_vol, vol_params,
           energy_params):
    site = site.astype(jnp.int32)
    conn = connectivity.astype(jnp.int32)
    site_flat = site.reshape(_NA, 1)
    site_i = site.reshape(_NA // 32, 32)
    offs = (jnp.arange(_B, dtype=jnp.int32) * _N)[:, None]
    dst_g = (conn[:, :, 0] + offs).reshape(_NE)
    src_g = (conn[:, :, 1] + offs).reshape(_NE)
    idx_cat = jnp.concatenate([src_g, dst_g]).reshape(2 * _NE // 128, 128)
    dstT = dst_g.reshape(_NBLK, 1, _EBLK)

    pred_vol = _gnn_pass(site_flat, site_i, distance, dstT, idx_cat,
                         vol_params)
    dist2 = distance * jnp.power(pred_vol / input_vol, 1.0 / 3.0)
    pred_energy = _gnn_pass(site_flat, site_i, dist2, dstT, idx_cat,
                            energy_params)
    return pred_vol, pred_energy
